# Initial kernel scaffold; baseline (speedup 1.0000x reference)
import jax
import jax.numpy as jnp
from jax.experimental import pallas as pl


def kernel(x, mask, W1, b1, W2, b2):
    raise NotImplementedError("write your pallas kernel here")


# speedup vs baseline: 10.4407x; 10.4407x over previous
import jax
import jax.numpy as jnp
from jax.experimental import pallas as pl
from jax.experimental.pallas import tpu as pltpu

_K = 8
_TOTAL = 3 * _K + 1
_TAIL = 3.0
_MIN_V = 1e-3
_MIN_D = 1e-3
_BB = 256


def _norm_edges(us):
    m = us[0]
    for u in us[1:]:
        m = jnp.maximum(m, u)
    es = [jnp.exp(u - m) for u in us]
    s = es[0]
    for e in es[1:]:
        s = s + e
    scale = (1.0 - _MIN_V * _K) / s
    ws = [_MIN_V + e * scale for e in es]
    edges = [jnp.full_like(us[0], -_TAIL)]
    cum = None
    for w in ws:
        cum = w if cum is None else cum + w
        edges.append((cum - 0.5) * (2.0 * _TAIL))
    return edges, ws


def _body(x_ref, mask_ref, w1_ref, b1_ref, w2_ref, b2_ref, out_ref, ld_ref):
    x = x_ref[...]
    mask = mask_ref[...]
    one_m = 1.0 - mask
    xs = x * mask
    xd = x * one_m

    h = jnp.dot(xs, w1_ref[...], preferred_element_type=jnp.float32) + b1_ref[...]
    h = jnp.maximum(h, 0.0)
    p = jnp.dot(h, w2_ref[...], preferred_element_type=jnp.float32) + b2_ref[...]

    D = x.shape[1]
    sl = lambda t: p[:, t * D:(t + 1) * D]

    edges_w, ws = _norm_edges([sl(t) for t in range(_K)])
    edges_h, hs = _norm_edges([sl(_K + t) for t in range(_K)])
    ds = [jax.nn.softplus(sl(2 * _K + t)) + _MIN_D for t in range(_K + 1)]
    edges_w[_K] = edges_w[_K] + 1e-6

    cnt = jnp.zeros_like(xd)
    for e in edges_w:
        cnt = cnt + jnp.where(xd >= e, 1.0, 0.0)
    idx = jnp.clip(cnt - 1.0, 0.0, float(_K - 1))

    cw = edges_w[0]
    ww = ws[0]
    ch = edges_h[0]
    hh = hs[0]
    di = ds[0]
    di1 = ds[1]
    for k in range(1, _K):
        selk = idx >= float(k)
        cw = jnp.where(selk, edges_w[k], cw)
        ww = jnp.where(selk, ws[k], ww)
        ch = jnp.where(selk, edges_h[k], ch)
        hh = jnp.where(selk, hs[k], hh)
        di = jnp.where(selk, ds[k], di)
        di1 = jnp.where(selk, ds[k + 1], di1)
    di = jnp.clip(di, _MIN_D, 1000.0)
    di1 = jnp.clip(di1, _MIN_D, 1000.0)

    inv_ww = 1.0 / ww
    delta = hh * inv_ww
    theta = jnp.clip((xd - cw) * inv_ww, 0.0, 1.0)
    t1m = theta * (1.0 - theta)
    th2 = theta * theta
    num_term = di1 * th2 + delta * t1m
    den = jnp.maximum(delta + (di + di1 - 2.0 * delta) * t1m, 1e-6)
    inv_den = 1.0 / den
    spline_out = ch + delta * num_term * inv_den * ww
    omt = 1.0 - theta
    deriv_num = (delta * delta) * (di1 * th2 + 2.0 * delta * t1m + di * (omt * omt))
    spline_ld = jnp.log(jnp.maximum(deriv_num * (inv_den * inv_den), 1e-12))

    in_range = (xd >= -_TAIL) & (xd <= _TAIL)
    new_dyn = jnp.where(in_range, spline_out, xd)
    ldet = jnp.where(in_range, spline_ld, 0.0)

    out_ref[...] = xs + new_dyn * one_m
    ld_ref[...] = jnp.sum(ldet * one_m, axis=1, keepdims=True)


def kernel(x, mask, W1, b1, W2, b2):
    B, D = x.shape
    H = W1.shape[1]
    W2p = W2.reshape(H, D, _TOTAL).transpose(0, 2, 1).reshape(H, _TOTAL * D)
    b2p = b2.reshape(D, _TOTAL).T.reshape(1, _TOTAL * D)
    mask2 = mask.reshape(1, D)
    b1r = b1.reshape(1, H)

    grid = (B // _BB,)
    out, ld = pl.pallas_call(
        _body,
        grid=grid,
        in_specs=[
            pl.BlockSpec((_BB, D), lambda i: (i, 0)),
            pl.BlockSpec((1, D), lambda i: (0, 0)),
            pl.BlockSpec((D, H), lambda i: (0, 0)),
            pl.BlockSpec((1, H), lambda i: (0, 0)),
            pl.BlockSpec((H, _TOTAL * D), lambda i: (0, 0)),
            pl.BlockSpec((1, _TOTAL * D), lambda i: (0, 0)),
        ],
        out_specs=[
            pl.BlockSpec((_BB, D), lambda i: (i, 0)),
            pl.BlockSpec((_BB, 1), lambda i: (i, 0)),
        ],
        out_shape=[
            jax.ShapeDtypeStruct((B, D), jnp.float32),
            jax.ShapeDtypeStruct((B, 1), jnp.float32),
        ],
        compiler_params=pltpu.CompilerParams(
            dimension_semantics=("arbitrary",),
            vmem_limit_bytes=56 * 1024 * 1024,
        ),
        name="spline_coupling_fused",
    )(x, mask2, W1, b1r, W2p, b2p)
    return out, ld.reshape(B)


# speedup vs baseline: 10.6042x; 1.0157x over previous
import jax
import jax.numpy as jnp
from jax.experimental import pallas as pl
from jax.experimental.pallas import tpu as pltpu

_K = 8
_TOTAL = 3 * _K + 1
_TAIL = 3.0
_MIN_V = 1e-3
_MIN_D = 1e-3
_BB = 256


def _norm_edges(us):
    m = us[0]
    for u in us[1:]:
        m = jnp.maximum(m, u)
    es = [jnp.exp(u - m) for u in us]
    s = es[0]
    for e in es[1:]:
        s = s + e
    scale = (1.0 - _MIN_V * _K) / s
    ws = [_MIN_V + e * scale for e in es]
    edges = [jnp.full_like(us[0], -_TAIL)]
    cum = None
    for w in ws:
        cum = w if cum is None else cum + w
        edges.append((cum - 0.5) * (2.0 * _TAIL))
    return edges, ws


def _body(x_ref, mask_ref, w1_ref, b1_ref, w2_ref, b2_ref, out_ref, ld_ref):
    x = x_ref[...]
    mask = mask_ref[...]
    one_m = 1.0 - mask
    xs = x * mask
    xd = x * one_m

    h = jnp.dot(xs.astype(jnp.bfloat16), w1_ref[...],
                preferred_element_type=jnp.float32) + b1_ref[...]
    h = jnp.maximum(h, 0.0)
    p = jnp.dot(h.astype(jnp.bfloat16), w2_ref[...],
                preferred_element_type=jnp.float32) + b2_ref[...]

    D = x.shape[1]
    sl = lambda t: p[:, t * D:(t + 1) * D]

    edges_w, ws = _norm_edges([sl(t) for t in range(_K)])
    edges_h, hs = _norm_edges([sl(_K + t) for t in range(_K)])
    ds = [jax.nn.softplus(sl(2 * _K + t)) + _MIN_D for t in range(_K + 1)]
    edges_w[_K] = edges_w[_K] + 1e-6

    cnt = jnp.zeros_like(xd)
    for e in edges_w:
        cnt = cnt + jnp.where(xd >= e, 1.0, 0.0)
    idx = jnp.clip(cnt - 1.0, 0.0, float(_K - 1))

    cw = edges_w[0]
    ww = ws[0]
    ch = edges_h[0]
    hh = hs[0]
    di = ds[0]
    di1 = ds[1]
    for k in range(1, _K):
        selk = idx >= float(k)
        cw = jnp.where(selk, edges_w[k], cw)
        ww = jnp.where(selk, ws[k], ww)
        ch = jnp.where(selk, edges_h[k], ch)
        hh = jnp.where(selk, hs[k], hh)
        di = jnp.where(selk, ds[k], di)
        di1 = jnp.where(selk, ds[k + 1], di1)
    di = jnp.clip(di, _MIN_D, 1000.0)
    di1 = jnp.clip(di1, _MIN_D, 1000.0)

    inv_ww = 1.0 / ww
    delta = hh * inv_ww
    theta = jnp.clip((xd - cw) * inv_ww, 0.0, 1.0)
    t1m = theta * (1.0 - theta)
    th2 = theta * theta
    num_term = di1 * th2 + delta * t1m
    den = jnp.maximum(delta + (di + di1 - 2.0 * delta) * t1m, 1e-6)
    inv_den = 1.0 / den
    spline_out = ch + delta * num_term * inv_den * ww
    omt = 1.0 - theta
    deriv_num = (delta * delta) * (di1 * th2 + 2.0 * delta * t1m + di * (omt * omt))
    spline_ld = jnp.log(jnp.maximum(deriv_num * (inv_den * inv_den), 1e-12))

    in_range = (xd >= -_TAIL) & (xd <= _TAIL)
    new_dyn = jnp.where(in_range, spline_out, xd)
    ldet = jnp.where(in_range, spline_ld, 0.0)

    out_ref[...] = xs + new_dyn * one_m
    ld_ref[...] = jnp.sum(ldet * one_m, axis=1, keepdims=True)


def kernel(x, mask, W1, b1, W2, b2):
    B, D = x.shape
    H = W1.shape[1]
    W2p = W2.reshape(H, D, _TOTAL).transpose(0, 2, 1).reshape(H, _TOTAL * D)
    W2p = W2p.astype(jnp.bfloat16)
    W1b = W1.astype(jnp.bfloat16)
    b2p = b2.reshape(D, _TOTAL).T.reshape(1, _TOTAL * D)
    mask2 = mask.reshape(1, D)
    b1r = b1.reshape(1, H)

    grid = (B // _BB,)
    out, ld = pl.pallas_call(
        _body,
        grid=grid,
        in_specs=[
            pl.BlockSpec((_BB, D), lambda i: (i, 0)),
            pl.BlockSpec((1, D), lambda i: (0, 0)),
            pl.BlockSpec((D, H), lambda i: (0, 0)),
            pl.BlockSpec((1, H), lambda i: (0, 0)),
            pl.BlockSpec((H, _TOTAL * D), lambda i: (0, 0)),
            pl.BlockSpec((1, _TOTAL * D), lambda i: (0, 0)),
        ],
        out_specs=[
            pl.BlockSpec((_BB, D), lambda i: (i, 0)),
            pl.BlockSpec((_BB, 1), lambda i: (i, 0)),
        ],
        out_shape=[
            jax.ShapeDtypeStruct((B, D), jnp.float32),
            jax.ShapeDtypeStruct((B, 1), jnp.float32),
        ],
        compiler_params=pltpu.CompilerParams(
            dimension_semantics=("arbitrary",),
            vmem_limit_bytes=56 * 1024 * 1024,
        ),
        name="spline_coupling_fused",
    )(x, mask2, W1b, b1r, W2p, b2p)
    return out, ld.reshape(B)
